# Initial kernel scaffold; baseline (speedup 1.0000x reference)
import jax
import jax.numpy as jnp
from jax.experimental import pallas as pl


def kernel(h_node, h_edge, edge_index, relative_vec, distance, edge_time, left_W1, left_b1, left_W2, left_b2, right_W1, right_b1, right_W2, right_b2, bond_Wb, bond_Wn, inter_W1, inter_b1, inter_W2, inter_b2, gate_W1, gate_b1, gate_W2, gate_b2):
    raise NotImplementedError("write your pallas kernel here")


# speedup vs baseline: 1.7916x; 1.7916x over previous
import functools

import jax
import jax.numpy as jnp
from jax import lax
from jax.experimental import pallas as pl
from jax.experimental.pallas import tpu as pltpu
from jax.experimental.pallas import tpu_sc as plsc

N = 10000
E = 320000
NODE_DIM = 128
EDGE_DIM = 16
HIDDEN_DIM = 128

NC, NS = 2, 16
NW = NC * NS
EPW = E // NW
CHUNK = 2000
STEPS = EPW // CHUNK
GB = 80
GPC = CHUNK // GB
NPS = N // NS

_SC_MESH = dict(core_axis_name="c", subcore_axis_name="s")


def _proj_body(hn, lW1, lb1, lW2, lb2, rW1, rb1, rW2, rb2, lout, rout):
    h = hn[...]
    l1 = jnp.maximum(h @ lW1[...] + lb1[...], 0.0)
    lout[...] = l1 @ lW2[...] + lb2[...]
    r1 = jnp.maximum(h @ rW1[...] + rb1[...], 0.0)
    rout[...] = r1 @ rW2[...] + rb2[...]


def _node_proj(h_node, lW1, lb1, lW2, lb2, rW1, rb1, rW2, rb2):
    BN = 2000
    grid = (N // BN,)
    full = lambda shape: pl.BlockSpec(shape, lambda i: (0, 0))
    return pl.pallas_call(
        _proj_body,
        grid=grid,
        in_specs=[
            pl.BlockSpec((BN, NODE_DIM), lambda i: (i, 0)),
            full((NODE_DIM, HIDDEN_DIM)), full((1, HIDDEN_DIM)),
            full((HIDDEN_DIM, EDGE_DIM)), full((1, EDGE_DIM)),
            full((NODE_DIM, HIDDEN_DIM)), full((1, HIDDEN_DIM)),
            full((HIDDEN_DIM, EDGE_DIM)), full((1, EDGE_DIM)),
        ],
        out_specs=(pl.BlockSpec((BN, EDGE_DIM), lambda i: (i, 0)),
                   pl.BlockSpec((BN, EDGE_DIM), lambda i: (i, 0))),
        out_shape=(jax.ShapeDtypeStruct((N, EDGE_DIM), jnp.float32),
                   jax.ShapeDtypeStruct((N, EDGE_DIM), jnp.float32)),
    )(h_node, lW1, lb1, lW2, lb2, rW1, rb1, rW2, rb2)


@functools.partial(
    pl.kernel,
    out_type=(jax.ShapeDtypeStruct((E, EDGE_DIM), jnp.float32),
              jax.ShapeDtypeStruct((E, EDGE_DIM), jnp.float32)),
    mesh=plsc.VectorSubcoreMesh(**_SC_MESH),
    compiler_params=pltpu.CompilerParams(use_tc_tiling_on_sc=False),
    scratch_types=[
        pltpu.VMEM((CHUNK,), jnp.int32),
        pltpu.VMEM((CHUNK,), jnp.int32),
        pltpu.VMEM((CHUNK, EDGE_DIM), jnp.float32),
        pltpu.VMEM((CHUNK, EDGE_DIM), jnp.float32),
        pltpu.SemaphoreType.DMA,
    ],
)
def _sc_gather(ltab, rtab, li_hbm, ri_hbm, lout, rout, liv, riv, lrows, rrows, sem):
    wid = lax.axis_index("s") * NC + lax.axis_index("c")
    base0 = wid * EPW

    def step(i, carry):
        base = base0 + i * CHUNK
        pltpu.sync_copy(li_hbm.at[pl.ds(base, CHUNK)], liv)
        pltpu.sync_copy(ri_hbm.at[pl.ds(base, CHUNK)], riv)
        descs = []
        for j in range(GPC):
            sl = pl.ds(j * GB, GB)
            descs.append(pltpu.async_copy(ltab.at[liv.at[sl]], lrows.at[sl], sem))
            descs.append(pltpu.async_copy(rtab.at[riv.at[sl]], rrows.at[sl], sem))
        for dsc in descs:
            dsc.wait()
        pltpu.sync_copy(lrows, lout.at[pl.ds(base, CHUNK)])
        pltpu.sync_copy(rrows, rout.at[pl.ds(base, CHUNK)])
        return carry

    lax.fori_loop(0, STEPS, step, 0)


def _edge_body(he, lf, rf, t, rel, dist, Wb, Wn, iW1, ib1, iW2r, ib2,
               gWe, gWn, gWt, gb1, gW2r, gb2, out):
    nfi = lf[...] * rf[...]
    bond = he[...] @ Wb[...]
    node = nfi @ Wn[...]
    x = bond * node
    h1 = jnp.maximum(x @ iW1[...] + ib1[...], 0.0)
    inter = jnp.sum(h1 * iW2r[...], axis=1, keepdims=True) + ib2[...]
    g1 = he[...] @ gWe[...] + nfi @ gWn[...] + t[...] * gWt[...] + gb1[...]
    g1 = jnp.maximum(g1, 0.0)
    gate = jnp.sum(g1 * gW2r[...], axis=1, keepdims=True) + gb2[...]
    w = inter * jax.nn.sigmoid(gate)
    d = dist[...]
    f3 = w * rel[...] / d / (d + 1.0)
    out[...] = jnp.concatenate(
        [f3, jnp.zeros((f3.shape[0], EDGE_DIM - 3), jnp.float32)], axis=1)


def _edge_dense(he, lf, rf, t, rel, dist, Wb, Wn, iW1, ib1, iW2r, ib2,
                gWe, gWn, gWt, gb1, gW2r, gb2):
    BE = 1000
    grid = (E // BE,)
    blk = lambda w: pl.BlockSpec((BE, w), lambda i: (i, 0))
    full = lambda shape: pl.BlockSpec(shape, lambda i: (0, 0))
    return pl.pallas_call(
        _edge_body,
        grid=grid,
        in_specs=[
            blk(EDGE_DIM), blk(EDGE_DIM), blk(EDGE_DIM), blk(1), blk(3), blk(1),
            full((EDGE_DIM, NODE_DIM)), full((EDGE_DIM, NODE_DIM)),
            full((NODE_DIM, NODE_DIM)), full((1, NODE_DIM)),
            full((1, NODE_DIM)), full((1, 1)),
            full((EDGE_DIM, 32)), full((EDGE_DIM, 32)), full((1, 32)),
            full((1, 32)), full((1, 32)), full((1, 1)),
        ],
        out_specs=pl.BlockSpec((BE, EDGE_DIM), lambda i: (i, 0)),
        out_shape=jax.ShapeDtypeStruct((E, EDGE_DIM), jnp.float32),
    )(he, lf, rf, t, rel, dist, Wb, Wn, iW1, ib1, iW2r, ib2,
      gWe, gWn, gWt, gb1, gW2r, gb2)


@functools.partial(
    pl.kernel,
    out_type=jax.ShapeDtypeStruct((NC, N, EDGE_DIM), jnp.float32),
    mesh=plsc.VectorSubcoreMesh(**_SC_MESH),
    compiler_params=pltpu.CompilerParams(use_tc_tiling_on_sc=False),
    scratch_types=[
        pltpu.VMEM((GPC, GB), jnp.int32),
        pltpu.VMEM((CHUNK, EDGE_DIM), jnp.float32),
        pltpu.VMEM((NPS, EDGE_DIM), jnp.float32),
        pltpu.VMEM_SHARED((N, EDGE_DIM), jnp.float32),
    ],
)
def _sc_scatter(f_hbm, li2_hbm, out, idxv, frows, zbuf, acc):
    c = lax.axis_index("c")
    s = lax.axis_index("s")

    def zrow(i, carry):
        zbuf[i, :] = jnp.zeros((EDGE_DIM,), jnp.float32)
        return carry

    lax.fori_loop(0, NPS, zrow, 0)
    pltpu.sync_copy(zbuf, acc.at[pl.ds(s * NPS, NPS)])
    plsc.subcore_barrier()

    wid_in_core = s
    base0 = (c * NS + wid_in_core) * EPW
    rbase0 = base0 // GB

    def step(i, carry):
        base = base0 + i * CHUNK
        rbase = rbase0 + i * GPC
        pltpu.sync_copy(li2_hbm.at[pl.ds(rbase, GPC)], idxv)
        pltpu.sync_copy(f_hbm.at[pl.ds(base, CHUNK)], frows)
        for j in range(GPC):
            pltpu.sync_copy(frows.at[pl.ds(j * GB, GB)], acc.at[idxv.at[j]],
                            add=True)
        return carry

    lax.fori_loop(0, STEPS, step, 0)
    plsc.subcore_barrier()
    pltpu.sync_copy(acc.at[pl.ds(s * NPS, NPS)], out.at[c, pl.ds(s * NPS, NPS)])


def _comb_body(p, o):
    o[...] = (p[0] + p[1])[:, :3]


def _combine(partials):
    BN = 2000
    return pl.pallas_call(
        _comb_body,
        grid=(N // BN,),
        in_specs=[pl.BlockSpec((NC, BN, EDGE_DIM), lambda i: (0, i, 0))],
        out_specs=pl.BlockSpec((BN, 3), lambda i: (i, 0)),
        out_shape=jax.ShapeDtypeStruct((N, 3), jnp.float32),
    )(partials)


def kernel(h_node, h_edge, edge_index, relative_vec, distance, edge_time,
           left_W1, left_b1, left_W2, left_b2,
           right_W1, right_b1, right_W2, right_b2,
           bond_Wb, bond_Wn,
           inter_W1, inter_b1, inter_W2, inter_b2,
           gate_W1, gate_b1, gate_W2, gate_b2):
    li = edge_index[0]
    ri = edge_index[1]

    lproj, rproj = _node_proj(
        h_node,
        left_W1, left_b1.reshape(1, -1), left_W2, left_b2.reshape(1, -1),
        right_W1, right_b1.reshape(1, -1), right_W2, right_b2.reshape(1, -1))

    lf, rf = _sc_gather(lproj, rproj, li, ri)

    force = _edge_dense(
        h_edge, lf, rf, edge_time, relative_vec, distance.reshape(E, 1),
        bond_Wb, bond_Wn,
        inter_W1, inter_b1.reshape(1, -1), inter_W2.reshape(1, -1),
        inter_b2.reshape(1, 1),
        gate_W1[:EDGE_DIM], gate_W1[EDGE_DIM:2 * EDGE_DIM],
        gate_W1[2 * EDGE_DIM:], gate_b1.reshape(1, -1),
        gate_W2.reshape(1, -1), gate_b2.reshape(1, 1))

    partials = _sc_scatter(force, li.reshape(E // GB, GB))
    return _combine(partials)


# speedup vs baseline: 7.1167x; 3.9722x over previous
import functools

import jax
import jax.numpy as jnp
from jax import lax
from jax.experimental import pallas as pl
from jax.experimental.pallas import tpu as pltpu
from jax.experimental.pallas import tpu_sc as plsc

N = 10000
E = 320000
NH = 2
EH = E // NH
NODE_DIM = 128
EDGE_DIM = 16
HIDDEN_DIM = 128

NC, NS = 2, 16
NW = NC * NS
EPW = EH // NW
CHUNK = 1000
STEPS = EPW // CHUNK
GB = 40
GPC = CHUNK // GB

_SC_MESH = dict(core_axis_name="c", subcore_axis_name="s")


def _proj_body(hn, lW1, lb1, lW2, lb2, rW1, rb1, rW2, rb2, lout, rout):
    h = hn[...]
    l1 = jnp.maximum(h @ lW1[...] + lb1[...], 0.0)
    lout[...] = l1 @ lW2[...] + lb2[...]
    r1 = jnp.maximum(h @ rW1[...] + rb1[...], 0.0)
    rout[...] = r1 @ rW2[...] + rb2[...]


def _node_proj(h_node, lW1, lb1, lW2, lb2, rW1, rb1, rW2, rb2):
    BN = 2000
    grid = (N // BN,)
    full = lambda shape: pl.BlockSpec(shape, lambda i: (0, 0))
    return pl.pallas_call(
        _proj_body,
        grid=grid,
        in_specs=[
            pl.BlockSpec((BN, NODE_DIM), lambda i: (i, 0)),
            full((NODE_DIM, HIDDEN_DIM)), full((1, HIDDEN_DIM)),
            full((HIDDEN_DIM, EDGE_DIM)), full((1, EDGE_DIM)),
            full((NODE_DIM, HIDDEN_DIM)), full((1, HIDDEN_DIM)),
            full((HIDDEN_DIM, EDGE_DIM)), full((1, EDGE_DIM)),
        ],
        out_specs=(pl.BlockSpec((BN, EDGE_DIM), lambda i: (i, 0)),
                   pl.BlockSpec((BN, EDGE_DIM), lambda i: (i, 0))),
        out_shape=(jax.ShapeDtypeStruct((N, EDGE_DIM), jnp.float32),
                   jax.ShapeDtypeStruct((N, EDGE_DIM), jnp.float32)),
    )(h_node, lW1, lb1, lW2, lb2, rW1, rb1, rW2, rb2)


def _make_gather(off):
    @functools.partial(
        pl.kernel,
        out_type=jax.ShapeDtypeStruct((EH, EDGE_DIM), jnp.float32),
        mesh=plsc.VectorSubcoreMesh(**_SC_MESH),
        compiler_params=pltpu.CompilerParams(use_tc_tiling_on_sc=False),
        scratch_types=[
            pltpu.VMEM((CHUNK,), jnp.int32),
            pltpu.VMEM((CHUNK,), jnp.int32),
            pltpu.VMEM((CHUNK, EDGE_DIM), jnp.float32),
            pltpu.VMEM((CHUNK, EDGE_DIM), jnp.float32),
            pltpu.SemaphoreType.DMA,
        ],
    )
    def gather(ltab, rtab, li_hbm, ri_hbm, nout, liv, riv, lrows, rrows, sem):
        wid = lax.axis_index("s") * NC + lax.axis_index("c")
        base0 = wid * EPW

        def step(i, carry):
            base = base0 + i * CHUNK
            pltpu.sync_copy(li_hbm.at[pl.ds(off + base, CHUNK)], liv)
            pltpu.sync_copy(ri_hbm.at[pl.ds(off + base, CHUNK)], riv)
            descs = []
            for j in range(GPC):
                sl = pl.ds(j * GB, GB)
                descs.append(
                    pltpu.async_copy(ltab.at[liv.at[sl]], lrows.at[sl], sem))
                descs.append(
                    pltpu.async_copy(rtab.at[riv.at[sl]], rrows.at[sl], sem))

            for j in range(GPC):
                descs[2 * j].wait()
                descs[2 * j + 1].wait()

                def mulrow(g, carry2, j=j):
                    r = j * GB + g * 8
                    for u in range(8):
                        lrows[r + u, :] = lrows[r + u, :] * rrows[r + u, :]
                    return carry2

                lax.fori_loop(0, GB // 8, mulrow, 0)
            pltpu.sync_copy(lrows, nout.at[pl.ds(base, CHUNK)])
            return carry

        lax.fori_loop(0, STEPS, step, 0)

    return gather


_gather_half = [_make_gather(0), _make_gather(EH)]


def _bmm(a, b):
    return jnp.dot(a.astype(jnp.bfloat16), b.astype(jnp.bfloat16),
                   preferred_element_type=jnp.float32)


def _edge_body(heT, nfiT, tT, relT, dT, Wcat, iW1T, ib1, iW2T, ib2,
               gWtT, gb1, gW2T, gb2, out):
    he = heT[...].astype(jnp.bfloat16)
    nfi = nfiT[...]
    u = jnp.concatenate([he, nfi], axis=0)
    y = _bmm(Wcat[...], u)
    bond = y[:NODE_DIM]
    node = y[NODE_DIM:2 * NODE_DIM]
    g1p = y[2 * NODE_DIM:]
    x = bond * node
    h1 = jnp.maximum(_bmm(iW1T[...], x) + ib1[...], 0.0)
    inter = _bmm(iW2T[...], h1) + ib2[...]
    g1 = g1p + gWtT[...] * tT[...] + gb1[...]
    g1 = jnp.maximum(g1, 0.0)
    gate = _bmm(gW2T[...], g1) + gb2[...]
    w = inter * jax.nn.sigmoid(gate)
    d = dT[...]
    out[...] = w * relT[...] / d / (d + 1.0)


BE = 3200


def _edge_dense(half, heT, nfiT, tT, relT, dT, Wcat, iW1T, ib1, iW2T, ib2,
                gWtT, gb1, gW2T, gb2):
    goff = half * (EH // BE)
    grid = (EH // BE,)
    blk = lambda r: pl.BlockSpec((r, BE), lambda i: (0, i + goff))
    full = lambda shape: pl.BlockSpec(shape, lambda i: (0, 0))
    return pl.pallas_call(
        _edge_body,
        grid=grid,
        in_specs=[
            blk(EDGE_DIM),
            pl.BlockSpec((EDGE_DIM, BE), lambda i: (0, i)),
            blk(1), blk(3), blk(1),
            full((2 * NODE_DIM + 32, 2 * EDGE_DIM)),
            full((NODE_DIM, NODE_DIM)), full((NODE_DIM, 1)),
            full((1, NODE_DIM)), full((1, 1)),
            full((32, 1)), full((32, 1)), full((1, 32)), full((1, 1)),
        ],
        out_specs=pl.BlockSpec((3, BE), lambda i: (0, i)),
        out_shape=jax.ShapeDtypeStruct((3, EH), jnp.float32),
    )(heT, nfiT, tT, relT, dT, Wcat, iW1T, ib1, iW2T, ib2,
      gWtT, gb1, gW2T, gb2)


NZC = 2000
NZS = N // NZC


def _make_scatter(off):
    roff = off // GB

    @functools.partial(
        pl.kernel,
        out_type=jax.ShapeDtypeStruct((NC, 3, N), jnp.float32),
        mesh=plsc.VectorSubcoreMesh(**_SC_MESH),
        compiler_params=pltpu.CompilerParams(use_tc_tiling_on_sc=False),
        scratch_types=[
            pltpu.VMEM((GPC, GB), jnp.int32),
            pltpu.VMEM((3, CHUNK), jnp.float32),
            pltpu.VMEM((NZC,), jnp.float32),
            pltpu.VMEM_SHARED((N,), jnp.float32),
            pltpu.VMEM_SHARED((N,), jnp.float32),
            pltpu.VMEM_SHARED((N,), jnp.float32),
            pltpu.SemaphoreType.DMA,
        ],
    )
    def scatter(fT_hbm, li2_hbm, out, idxv, fcomp, zbuf, accx, accy, accz, sem):
        c = lax.axis_index("c")
        s = lax.axis_index("s")
        planes = [accx, accy, accz]

        def zrow(i, carry):
            zbuf[pl.ds(i * 16, 16)] = jnp.zeros((16,), jnp.float32)
            return carry

        lax.fori_loop(0, NZC // 16, zrow, 0)
        for comp in range(3):
            @pl.when(s == comp)
            def _(comp=comp):
                for j in range(NZS):
                    pltpu.sync_copy(zbuf, planes[comp].at[pl.ds(j * NZC, NZC)])

        plsc.subcore_barrier()

        base0 = (c * NS + s) * EPW
        rbase0 = roff + base0 // GB

        def step(i, carry):
            base = base0 + i * CHUNK
            rbase = rbase0 + i * GPC
            pltpu.sync_copy(li2_hbm.at[pl.ds(rbase, GPC)], idxv)
            for comp in range(3):
                pltpu.sync_copy(fT_hbm.at[comp, pl.ds(base, CHUNK)],
                                fcomp.at[comp])
            descs = []
            for comp in range(3):
                for j in range(GPC):
                    descs.append(pltpu.async_copy(
                        fcomp.at[comp, pl.ds(j * GB, GB)],
                        planes[comp].at[idxv.at[j]], sem, add=True))
            for dsc in descs:
                dsc.wait()
            return carry

        lax.fori_loop(0, STEPS, step, 0)
        plsc.subcore_barrier()
        for comp in range(3):
            for j in range(NZS):
                @pl.when(s == comp * NZS + j)
                def _(comp=comp, j=j):
                    pltpu.sync_copy(planes[comp].at[pl.ds(j * NZC, NZC)],
                                    out.at[c, comp, pl.ds(j * NZC, NZC)])

    return scatter


_scatter_half = [_make_scatter(0), _make_scatter(EH)]


def _comb_body(p0, p1, o):
    o[...] = (p0[0] + p0[1]) + (p1[0] + p1[1])


def _combine(part0, part1):
    return pl.pallas_call(
        _comb_body,
        out_shape=jax.ShapeDtypeStruct((3, N), jnp.float32),
    )(part0, part1)


def kernel(h_node, h_edge, edge_index, relative_vec, distance, edge_time,
           left_W1, left_b1, left_W2, left_b2,
           right_W1, right_b1, right_W2, right_b2,
           bond_Wb, bond_Wn,
           inter_W1, inter_b1, inter_W2, inter_b2,
           gate_W1, gate_b1, gate_W2, gate_b2):
    li = edge_index[0]
    ri = edge_index[1]
    lproj, rproj = _node_proj(
        h_node,
        left_W1, left_b1.reshape(1, -1), left_W2, left_b2.reshape(1, -1),
        right_W1, right_b1.reshape(1, -1), right_W2, right_b2.reshape(1, -1))

    z16 = jnp.zeros((NODE_DIM, EDGE_DIM), jnp.float32)
    Wcat = jnp.concatenate([
        jnp.concatenate([bond_Wb.T, z16], axis=1),
        jnp.concatenate([z16, bond_Wn.T], axis=1),
        jnp.concatenate([gate_W1[:EDGE_DIM].T,
                         gate_W1[EDGE_DIM:2 * EDGE_DIM].T], axis=1),
    ], axis=0)

    heT = h_edge.T
    tT = edge_time.reshape(1, E)
    relT = relative_vec.T
    dT = distance.reshape(1, E)
    li2 = li.reshape(E // GB, GB)

    parts = []
    for half in range(NH):
        nfi = _gather_half[half](lproj, rproj, li, ri)
        nfiT = nfi.astype(jnp.bfloat16).T
        forceT = _edge_dense(
            half, heT, nfiT, tT, relT, dT,
            Wcat,
            inter_W1.T, inter_b1.reshape(-1, 1), inter_W2.reshape(1, -1),
            inter_b2.reshape(1, 1),
            gate_W1[2 * EDGE_DIM:].T, gate_b1.reshape(-1, 1),
            gate_W2.reshape(1, -1), gate_b2.reshape(1, 1))
        parts.append(_scatter_half[half](forceT, li2))

    return _combine(parts[0], parts[1]).T


# speedup vs baseline: 7.6482x; 1.0747x over previous
import functools

import jax
import jax.numpy as jnp
from jax import lax
from jax.experimental import pallas as pl
from jax.experimental.pallas import tpu as pltpu
from jax.experimental.pallas import tpu_sc as plsc

N = 10000
E = 320000
NODE_DIM = 128
EDGE_DIM = 16
HIDDEN_DIM = 128

NC, NS = 2, 16
NW = NC * NS
EPW = E // NW
CHUNK = 2000
STEPS = EPW // CHUNK
GB = 80
GPC = CHUNK // GB
NPS = N // NS

_SC_MESH = dict(core_axis_name="c", subcore_axis_name="s")


def _proj_body(hn, lW1, lb1, lW2, lb2, rW1, rb1, rW2, rb2, lout, rout):
    h = hn[...]
    l1 = jnp.maximum(h @ lW1[...] + lb1[...], 0.0)
    lout[...] = l1 @ lW2[...] + lb2[...]
    r1 = jnp.maximum(h @ rW1[...] + rb1[...], 0.0)
    rout[...] = r1 @ rW2[...] + rb2[...]


def _node_proj(h_node, lW1, lb1, lW2, lb2, rW1, rb1, rW2, rb2):
    BN = 2000
    grid = (N // BN,)
    full = lambda shape: pl.BlockSpec(shape, lambda i: (0, 0))
    return pl.pallas_call(
        _proj_body,
        grid=grid,
        in_specs=[
            pl.BlockSpec((BN, NODE_DIM), lambda i: (i, 0)),
            full((NODE_DIM, HIDDEN_DIM)), full((1, HIDDEN_DIM)),
            full((HIDDEN_DIM, EDGE_DIM)), full((1, EDGE_DIM)),
            full((NODE_DIM, HIDDEN_DIM)), full((1, HIDDEN_DIM)),
            full((HIDDEN_DIM, EDGE_DIM)), full((1, EDGE_DIM)),
        ],
        out_specs=(pl.BlockSpec((BN, EDGE_DIM), lambda i: (i, 0)),
                   pl.BlockSpec((BN, EDGE_DIM), lambda i: (i, 0))),
        out_shape=(jax.ShapeDtypeStruct((N, EDGE_DIM), jnp.float32),
                   jax.ShapeDtypeStruct((N, EDGE_DIM), jnp.float32)),
    )(h_node, lW1, lb1, lW2, lb2, rW1, rb1, rW2, rb2)


@functools.partial(
    pl.kernel,
    out_type=jax.ShapeDtypeStruct((E, EDGE_DIM), jnp.float32),
    mesh=plsc.VectorSubcoreMesh(**_SC_MESH),
    compiler_params=pltpu.CompilerParams(use_tc_tiling_on_sc=False),
    scratch_types=[
        pltpu.VMEM((CHUNK,), jnp.int32),
        pltpu.VMEM((CHUNK,), jnp.int32),
        pltpu.VMEM((CHUNK, EDGE_DIM), jnp.float32),
        pltpu.VMEM((CHUNK, EDGE_DIM), jnp.float32),
        pltpu.SemaphoreType.DMA,
    ],
)
def _sc_gather(ltab, rtab, li_hbm, ri_hbm, nout, liv, riv, lrows, rrows, sem):
    wid = lax.axis_index("s") * NC + lax.axis_index("c")
    base0 = wid * EPW

    def step(i, carry):
        base = base0 + i * CHUNK
        pltpu.sync_copy(li_hbm.at[pl.ds(base, CHUNK)], liv)
        pltpu.sync_copy(ri_hbm.at[pl.ds(base, CHUNK)], riv)
        descs = []
        for j in range(GPC):
            sl = pl.ds(j * GB, GB)
            descs.append(pltpu.async_copy(ltab.at[liv.at[sl]], lrows.at[sl], sem))
            descs.append(pltpu.async_copy(rtab.at[riv.at[sl]], rrows.at[sl], sem))

        for j in range(GPC):
            descs[2 * j].wait()
            descs[2 * j + 1].wait()

            def mulrow(g, carry2, j=j):
                r = j * GB + g * 8
                for u in range(8):
                    lrows[r + u, :] = lrows[r + u, :] * rrows[r + u, :]
                return carry2

            lax.fori_loop(0, GB // 8, mulrow, 0)
        pltpu.sync_copy(lrows, nout.at[pl.ds(base, CHUNK)])
        return carry

    lax.fori_loop(0, STEPS, step, 0)


def _bmm(a, b):
    return jnp.dot(a.astype(jnp.bfloat16), b.astype(jnp.bfloat16),
                   preferred_element_type=jnp.float32)


def _edge_body(heT, nfiT, tT, relT, dT, Wcat, iW1T, ib1, iW2T, ib2,
               gWtT, gb1, gW2T, gb2, out):
    he = heT[...].astype(jnp.bfloat16)
    nfi = nfiT[...]
    u = jnp.concatenate([he, nfi], axis=0)
    y = _bmm(Wcat[...], u)
    bond = y[:NODE_DIM]
    node = y[NODE_DIM:2 * NODE_DIM]
    g1p = y[2 * NODE_DIM:]
    x = bond * node
    h1 = jnp.maximum(_bmm(iW1T[...], x) + ib1[...], 0.0)
    inter = _bmm(iW2T[...], h1) + ib2[...]
    g1 = g1p + gWtT[...] * tT[...] + gb1[...]
    g1 = jnp.maximum(g1, 0.0)
    gate = _bmm(gW2T[...], g1) + gb2[...]
    w = inter * jax.nn.sigmoid(gate)
    d = dT[...]
    out[...] = w * relT[...] / d / (d + 1.0)


def _edge_dense(heT, nfiT, tT, relT, dT, Wcat, iW1T, ib1, iW2T, ib2,
                gWtT, gb1, gW2T, gb2):
    BE = 6400
    grid = (E // BE,)
    blk = lambda r: pl.BlockSpec((r, BE), lambda i: (0, i))
    full = lambda shape: pl.BlockSpec(shape, lambda i: (0, 0))
    return pl.pallas_call(
        _edge_body,
        grid=grid,
        in_specs=[
            blk(EDGE_DIM), blk(EDGE_DIM), blk(1), blk(3), blk(1),
            full((2 * NODE_DIM + 32, 2 * EDGE_DIM)),
            full((NODE_DIM, NODE_DIM)), full((NODE_DIM, 1)),
            full((1, NODE_DIM)), full((1, 1)),
            full((32, 1)), full((32, 1)), full((1, 32)), full((1, 1)),
        ],
        out_specs=pl.BlockSpec((3, BE), lambda i: (0, i)),
        out_shape=jax.ShapeDtypeStruct((3, E), jnp.float32),
    )(heT, nfiT, tT, relT, dT, Wcat, iW1T, ib1, iW2T, ib2,
      gWtT, gb1, gW2T, gb2)


NZC = 2000
NZS = N // NZC


@functools.partial(
    pl.kernel,
    out_type=jax.ShapeDtypeStruct((NC, 3, N), jnp.float32),
    mesh=plsc.VectorSubcoreMesh(**_SC_MESH),
    compiler_params=pltpu.CompilerParams(use_tc_tiling_on_sc=False),
    scratch_types=[
        pltpu.VMEM((GPC, GB), jnp.int32),
        pltpu.VMEM((3, CHUNK), jnp.float32),
        pltpu.VMEM((NZC,), jnp.float32),
        pltpu.VMEM_SHARED((N,), jnp.float32),
        pltpu.VMEM_SHARED((N,), jnp.float32),
        pltpu.VMEM_SHARED((N,), jnp.float32),
        pltpu.SemaphoreType.DMA,
    ],
)
def _sc_scatter(fT_hbm, li2_hbm, out, idxv, fcomp, zbuf, accx, accy, accz, sem):
    c = lax.axis_index("c")
    s = lax.axis_index("s")
    planes = [accx, accy, accz]

    def zrow(i, carry):
        zbuf[pl.ds(i * 16, 16)] = jnp.zeros((16,), jnp.float32)
        return carry

    lax.fori_loop(0, NZC // 16, zrow, 0)
    for comp in range(3):
        @pl.when(s == comp)
        def _(comp=comp):
            for j in range(NZS):
                pltpu.sync_copy(zbuf, planes[comp].at[pl.ds(j * NZC, NZC)])

    plsc.subcore_barrier()

    base0 = (c * NS + s) * EPW
    rbase0 = base0 // GB

    def step(i, carry):
        base = base0 + i * CHUNK
        rbase = rbase0 + i * GPC
        pltpu.sync_copy(li2_hbm.at[pl.ds(rbase, GPC)], idxv)
        for comp in range(3):
            pltpu.sync_copy(fT_hbm.at[comp, pl.ds(base, CHUNK)],
                            fcomp.at[comp])
        descs = []
        for comp in range(3):
            for j in range(GPC):
                descs.append(pltpu.async_copy(
                    fcomp.at[comp, pl.ds(j * GB, GB)],
                    planes[comp].at[idxv.at[j]], sem, add=True))
        for dsc in descs:
            dsc.wait()
        return carry

    lax.fori_loop(0, STEPS, step, 0)
    plsc.subcore_barrier()
    for comp in range(3):
        for j in range(NZS):
            @pl.when(s == comp * NZS + j)
            def _(comp=comp, j=j):
                pltpu.sync_copy(planes[comp].at[pl.ds(j * NZC, NZC)],
                                out.at[c, comp, pl.ds(j * NZC, NZC)])


def _comb_body(p, o):
    o[...] = p[0] + p[1]


def _combine(partials):
    return pl.pallas_call(
        _comb_body,
        out_shape=jax.ShapeDtypeStruct((3, N), jnp.float32),
    )(partials)


def kernel(h_node, h_edge, edge_index, relative_vec, distance, edge_time,
           left_W1, left_b1, left_W2, left_b2,
           right_W1, right_b1, right_W2, right_b2,
           bond_Wb, bond_Wn,
           inter_W1, inter_b1, inter_W2, inter_b2,
           gate_W1, gate_b1, gate_W2, gate_b2):
    li = edge_index[0]
    ri = edge_index[1]
    lproj, rproj = _node_proj(
        h_node,
        left_W1, left_b1.reshape(1, -1), left_W2, left_b2.reshape(1, -1),
        right_W1, right_b1.reshape(1, -1), right_W2, right_b2.reshape(1, -1))

    nfi = _sc_gather(lproj, rproj, li, ri)

    z16 = jnp.zeros((NODE_DIM, EDGE_DIM), jnp.float32)
    Wcat = jnp.concatenate([
        jnp.concatenate([bond_Wb.T, z16], axis=1),
        jnp.concatenate([z16, bond_Wn.T], axis=1),
        jnp.concatenate([gate_W1[:EDGE_DIM].T,
                         gate_W1[EDGE_DIM:2 * EDGE_DIM].T], axis=1),
    ], axis=0)

    forceT = _edge_dense(
        h_edge.T, nfi.astype(jnp.bfloat16).T, edge_time.reshape(1, E),
        relative_vec.T,
        distance.reshape(1, E),
        Wcat,
        inter_W1.T, inter_b1.reshape(-1, 1), inter_W2.reshape(1, -1),
        inter_b2.reshape(1, 1),
        gate_W1[2 * EDGE_DIM:].T, gate_b1.reshape(-1, 1),
        gate_W2.reshape(1, -1), gate_b2.reshape(1, 1))

    partials = _sc_scatter(forceT, li.reshape(E // GB, GB))
    return _combine(partials).T


# speedup vs baseline: 7.7203x; 1.0094x over previous
import functools

import jax
import jax.numpy as jnp
from jax import lax
from jax.experimental import pallas as pl
from jax.experimental.pallas import tpu as pltpu
from jax.experimental.pallas import tpu_sc as plsc

N = 10000
E = 320000
NODE_DIM = 128
EDGE_DIM = 16
HIDDEN_DIM = 128

NC, NS = 2, 16
NW = NC * NS
EPW = E // NW
CHUNK = 2000
STEPS = EPW // CHUNK
GB = 80
GPC = CHUNK // GB
NPS = N // NS

_SC_MESH = dict(core_axis_name="c", subcore_axis_name="s")


def _proj_body(hn, lW1, lb1, lW2, lb2, rW1, rb1, rW2, rb2, lout, rout):
    h = hn[...]
    l1 = jnp.maximum(h @ lW1[...] + lb1[...], 0.0)
    lout[...] = l1 @ lW2[...] + lb2[...]
    r1 = jnp.maximum(h @ rW1[...] + rb1[...], 0.0)
    rout[...] = r1 @ rW2[...] + rb2[...]


def _node_proj(h_node, lW1, lb1, lW2, lb2, rW1, rb1, rW2, rb2):
    BN = 2000
    grid = (N // BN,)
    full = lambda shape: pl.BlockSpec(shape, lambda i: (0, 0))
    return pl.pallas_call(
        _proj_body,
        grid=grid,
        in_specs=[
            pl.BlockSpec((BN, NODE_DIM), lambda i: (i, 0)),
            full((NODE_DIM, HIDDEN_DIM)), full((1, HIDDEN_DIM)),
            full((HIDDEN_DIM, EDGE_DIM)), full((1, EDGE_DIM)),
            full((NODE_DIM, HIDDEN_DIM)), full((1, HIDDEN_DIM)),
            full((HIDDEN_DIM, EDGE_DIM)), full((1, EDGE_DIM)),
        ],
        out_specs=(pl.BlockSpec((BN, EDGE_DIM), lambda i: (i, 0)),
                   pl.BlockSpec((BN, EDGE_DIM), lambda i: (i, 0))),
        out_shape=(jax.ShapeDtypeStruct((N, EDGE_DIM), jnp.float32),
                   jax.ShapeDtypeStruct((N, EDGE_DIM), jnp.float32)),
    )(h_node, lW1, lb1, lW2, lb2, rW1, rb1, rW2, rb2)


@functools.partial(
    pl.kernel,
    out_type=jax.ShapeDtypeStruct((E, EDGE_DIM), jnp.float32),
    mesh=plsc.VectorSubcoreMesh(**_SC_MESH),
    compiler_params=pltpu.CompilerParams(use_tc_tiling_on_sc=False),
    scratch_types=[
        pltpu.VMEM((CHUNK,), jnp.int32),
        pltpu.VMEM((CHUNK,), jnp.int32),
        pltpu.VMEM((CHUNK, EDGE_DIM), jnp.float32),
        pltpu.VMEM((CHUNK, EDGE_DIM), jnp.float32),
        pltpu.SemaphoreType.DMA,
    ],
)
def _sc_gather(ltab, rtab, li_hbm, ri_hbm, nout, liv, riv, lrows, rrows, sem):
    wid = lax.axis_index("s") * NC + lax.axis_index("c")
    base0 = wid * EPW

    def step(i, carry):
        base = base0 + i * CHUNK
        pltpu.sync_copy(li_hbm.at[pl.ds(base, CHUNK)], liv)
        pltpu.sync_copy(ri_hbm.at[pl.ds(base, CHUNK)], riv)
        descs = []
        for j in range(GPC):
            sl = pl.ds(j * GB, GB)
            descs.append(pltpu.async_copy(ltab.at[liv.at[sl]], lrows.at[sl], sem))
            descs.append(pltpu.async_copy(rtab.at[riv.at[sl]], rrows.at[sl], sem))

        for j in range(GPC):
            descs[2 * j].wait()
            descs[2 * j + 1].wait()

            def mulrow(g, carry2, j=j):
                r = j * GB + g * 8
                for u in range(8):
                    lrows[r + u, :] = lrows[r + u, :] * rrows[r + u, :]
                return carry2

            lax.fori_loop(0, GB // 8, mulrow, 0)
        pltpu.sync_copy(lrows, nout.at[pl.ds(base, CHUNK)])
        return carry

    lax.fori_loop(0, STEPS, step, 0)


def _bmm(a, b):
    return jnp.dot(a.astype(jnp.bfloat16), b.astype(jnp.bfloat16),
                   preferred_element_type=jnp.float32)


def _edge_body(heT, nfiT, tT, relT, dT, Wcat, iW1T, ib1, iW2T, ib2,
               gWtT, gb1, gW2T, gb2, out):
    he = heT[...].astype(jnp.bfloat16)
    nfi = nfiT[...]
    u = jnp.concatenate([he, nfi], axis=0)
    y = _bmm(Wcat[...], u)
    bond = y[:NODE_DIM]
    node = y[NODE_DIM:2 * NODE_DIM]
    g1p = y[2 * NODE_DIM:]
    x = bond * node
    h1 = jnp.maximum(_bmm(iW1T[...], x) + ib1[...], 0.0)
    inter = _bmm(iW2T[...], h1) + ib2[...]
    g1 = g1p + gWtT[...] * tT[...] + gb1[...]
    g1 = jnp.maximum(g1, 0.0)
    gate = _bmm(gW2T[...], g1) + gb2[...]
    w = inter * jax.nn.sigmoid(gate)
    d = dT[...]
    out[...] = w * relT[...] / d / (d + 1.0)


def _edge_dense(heT, nfiT, tT, relT, dT, Wcat, iW1T, ib1, iW2T, ib2,
                gWtT, gb1, gW2T, gb2):
    BE = 12800
    grid = (E // BE,)
    blk = lambda r: pl.BlockSpec((r, BE), lambda i: (0, i))
    full = lambda shape: pl.BlockSpec(shape, lambda i: (0, 0))
    return pl.pallas_call(
        _edge_body,
        grid=grid,
        in_specs=[
            blk(EDGE_DIM), blk(EDGE_DIM), blk(1), blk(3), blk(1),
            full((2 * NODE_DIM + 32, 2 * EDGE_DIM)),
            full((NODE_DIM, NODE_DIM)), full((NODE_DIM, 1)),
            full((1, NODE_DIM)), full((1, 1)),
            full((32, 1)), full((32, 1)), full((1, 32)), full((1, 1)),
        ],
        out_specs=pl.BlockSpec((3, BE), lambda i: (0, i)),
        out_shape=jax.ShapeDtypeStruct((3, E), jnp.float32),
    )(heT, nfiT, tT, relT, dT, Wcat, iW1T, ib1, iW2T, ib2,
      gWtT, gb1, gW2T, gb2)


NZC = 2000
NZS = N // NZC


@functools.partial(
    pl.kernel,
    out_type=jax.ShapeDtypeStruct((NC, 3, N), jnp.float32),
    mesh=plsc.VectorSubcoreMesh(**_SC_MESH),
    compiler_params=pltpu.CompilerParams(use_tc_tiling_on_sc=False),
    scratch_types=[
        pltpu.VMEM((GPC, GB), jnp.int32),
        pltpu.VMEM((3, CHUNK), jnp.float32),
        pltpu.VMEM((NZC,), jnp.float32),
        pltpu.VMEM_SHARED((N,), jnp.float32),
        pltpu.VMEM_SHARED((N,), jnp.float32),
        pltpu.VMEM_SHARED((N,), jnp.float32),
        pltpu.SemaphoreType.DMA,
    ],
)
def _sc_scatter(fT_hbm, li2_hbm, out, idxv, fcomp, zbuf, accx, accy, accz, sem):
    c = lax.axis_index("c")
    s = lax.axis_index("s")
    planes = [accx, accy, accz]

    def zrow(i, carry):
        zbuf[pl.ds(i * 16, 16)] = jnp.zeros((16,), jnp.float32)
        return carry

    lax.fori_loop(0, NZC // 16, zrow, 0)
    for comp in range(3):
        @pl.when(s == comp)
        def _(comp=comp):
            for j in range(NZS):
                pltpu.sync_copy(zbuf, planes[comp].at[pl.ds(j * NZC, NZC)])

    plsc.subcore_barrier()

    base0 = (c * NS + s) * EPW
    rbase0 = base0 // GB

    def step(i, carry):
        base = base0 + i * CHUNK
        rbase = rbase0 + i * GPC
        pltpu.sync_copy(li2_hbm.at[pl.ds(rbase, GPC)], idxv)
        for comp in range(3):
            pltpu.sync_copy(fT_hbm.at[comp, pl.ds(base, CHUNK)],
                            fcomp.at[comp])
        descs = []
        for comp in range(3):
            for j in range(GPC):
                descs.append(pltpu.async_copy(
                    fcomp.at[comp, pl.ds(j * GB, GB)],
                    planes[comp].at[idxv.at[j]], sem, add=True))
        for dsc in descs:
            dsc.wait()
        return carry

    lax.fori_loop(0, STEPS, step, 0)
    plsc.subcore_barrier()
    for comp in range(3):
        for j in range(NZS):
            @pl.when(s == comp * NZS + j)
            def _(comp=comp, j=j):
                pltpu.sync_copy(planes[comp].at[pl.ds(j * NZC, NZC)],
                                out.at[c, comp, pl.ds(j * NZC, NZC)])


def _comb_body(p, o):
    o[...] = p[0] + p[1]


def _combine(partials):
    return pl.pallas_call(
        _comb_body,
        out_shape=jax.ShapeDtypeStruct((3, N), jnp.float32),
    )(partials)


def kernel(h_node, h_edge, edge_index, relative_vec, distance, edge_time,
           left_W1, left_b1, left_W2, left_b2,
           right_W1, right_b1, right_W2, right_b2,
           bond_Wb, bond_Wn,
           inter_W1, inter_b1, inter_W2, inter_b2,
           gate_W1, gate_b1, gate_W2, gate_b2):
    li = edge_index[0]
    ri = edge_index[1]
    lproj, rproj = _node_proj(
        h_node,
        left_W1, left_b1.reshape(1, -1), left_W2, left_b2.reshape(1, -1),
        right_W1, right_b1.reshape(1, -1), right_W2, right_b2.reshape(1, -1))

    nfi = _sc_gather(lproj, rproj, li, ri)

    z16 = jnp.zeros((NODE_DIM, EDGE_DIM), jnp.float32)
    Wcat = jnp.concatenate([
        jnp.concatenate([bond_Wb.T, z16], axis=1),
        jnp.concatenate([z16, bond_Wn.T], axis=1),
        jnp.concatenate([gate_W1[:EDGE_DIM].T,
                         gate_W1[EDGE_DIM:2 * EDGE_DIM].T], axis=1),
    ], axis=0)

    forceT = _edge_dense(
        h_edge.T, nfi.astype(jnp.bfloat16).T, edge_time.reshape(1, E),
        relative_vec.T,
        distance.reshape(1, E),
        Wcat,
        inter_W1.T, inter_b1.reshape(-1, 1), inter_W2.reshape(1, -1),
        inter_b2.reshape(1, 1),
        gate_W1[2 * EDGE_DIM:].T, gate_b1.reshape(-1, 1),
        gate_W2.reshape(1, -1), gate_b2.reshape(1, 1))

    partials = _sc_scatter(forceT, li.reshape(E // GB, GB))
    return _combine(partials).T


# speedup vs baseline: 7.7468x; 1.0034x over previous
import functools

import jax
import jax.numpy as jnp
from jax import lax
from jax.experimental import pallas as pl
from jax.experimental.pallas import tpu as pltpu
from jax.experimental.pallas import tpu_sc as plsc

N = 10000
E = 320000
NODE_DIM = 128
EDGE_DIM = 16
HIDDEN_DIM = 128

NC, NS = 2, 16
NW = NC * NS
EPW = E // NW
CHUNK = 2000
STEPS = EPW // CHUNK
GB = 80
GPC = CHUNK // GB
NPS = N // NS

_SC_MESH = dict(core_axis_name="c", subcore_axis_name="s")


def _proj_body(hn, lW1, lb1, lW2, lb2, rW1, rb1, rW2, rb2, lout, rout):
    h = hn[...]
    l1 = jnp.maximum(h @ lW1[...] + lb1[...], 0.0)
    lout[...] = l1 @ lW2[...] + lb2[...]
    r1 = jnp.maximum(h @ rW1[...] + rb1[...], 0.0)
    rout[...] = r1 @ rW2[...] + rb2[...]


def _node_proj(h_node, lW1, lb1, lW2, lb2, rW1, rb1, rW2, rb2):
    BN = 2000
    grid = (N // BN,)
    full = lambda shape: pl.BlockSpec(shape, lambda i: (0, 0))
    return pl.pallas_call(
        _proj_body,
        grid=grid,
        in_specs=[
            pl.BlockSpec((BN, NODE_DIM), lambda i: (i, 0)),
            full((NODE_DIM, HIDDEN_DIM)), full((1, HIDDEN_DIM)),
            full((HIDDEN_DIM, EDGE_DIM)), full((1, EDGE_DIM)),
            full((NODE_DIM, HIDDEN_DIM)), full((1, HIDDEN_DIM)),
            full((HIDDEN_DIM, EDGE_DIM)), full((1, EDGE_DIM)),
        ],
        out_specs=(pl.BlockSpec((BN, EDGE_DIM), lambda i: (i, 0)),
                   pl.BlockSpec((BN, EDGE_DIM), lambda i: (i, 0))),
        out_shape=(jax.ShapeDtypeStruct((N, EDGE_DIM), jnp.float32),
                   jax.ShapeDtypeStruct((N, EDGE_DIM), jnp.float32)),
    )(h_node, lW1, lb1, lW2, lb2, rW1, rb1, rW2, rb2)


@functools.partial(
    pl.kernel,
    out_type=jax.ShapeDtypeStruct((E, EDGE_DIM), jnp.float32),
    mesh=plsc.VectorSubcoreMesh(**_SC_MESH),
    compiler_params=pltpu.CompilerParams(use_tc_tiling_on_sc=False),
    scratch_types=[
        pltpu.VMEM((CHUNK,), jnp.int32),
        pltpu.VMEM((CHUNK,), jnp.int32),
        pltpu.VMEM((CHUNK, EDGE_DIM), jnp.float32),
        pltpu.VMEM((CHUNK, EDGE_DIM), jnp.float32),
        pltpu.SemaphoreType.DMA,
    ],
)
def _sc_gather(ltab, rtab, li_hbm, ri_hbm, nout, liv, riv, lrows, rrows, sem):
    wid = lax.axis_index("s") * NC + lax.axis_index("c")
    base0 = wid * EPW

    def step(i, carry):
        base = base0 + i * CHUNK
        pltpu.sync_copy(li_hbm.at[pl.ds(base, CHUNK)], liv)
        pltpu.sync_copy(ri_hbm.at[pl.ds(base, CHUNK)], riv)
        descs = []
        for j in range(GPC):
            sl = pl.ds(j * GB, GB)
            descs.append(pltpu.async_copy(ltab.at[liv.at[sl]], lrows.at[sl], sem))
            descs.append(pltpu.async_copy(rtab.at[riv.at[sl]], rrows.at[sl], sem))

        for j in range(GPC):
            descs[2 * j].wait()
            descs[2 * j + 1].wait()

            def mulrow(g, carry2, j=j):
                r = j * GB + g * 8
                for u in range(8):
                    lrows[r + u, :] = lrows[r + u, :] * rrows[r + u, :]
                return carry2

            lax.fori_loop(0, GB // 8, mulrow, 0)
        pltpu.sync_copy(lrows, nout.at[pl.ds(base, CHUNK)])
        return carry

    lax.fori_loop(0, STEPS, step, 0)


def _bmm(a, b):
    return jnp.dot(a.astype(jnp.bfloat16), b.astype(jnp.bfloat16),
                   preferred_element_type=jnp.float32)


def _edge_body(heT, nfiT, tT, relT, dT, Wcat, iW1T, ib1, iW2T, ib2,
               gWtT, gb1, gW2T, gb2, out):
    he = heT[...].astype(jnp.bfloat16)
    nfi = nfiT[...]
    u = jnp.concatenate([he, nfi], axis=0)
    y = _bmm(Wcat[...], u)
    bond = y[:NODE_DIM]
    node = y[NODE_DIM:2 * NODE_DIM]
    g1p = y[2 * NODE_DIM:]
    x = bond * node
    h1 = jnp.maximum(_bmm(iW1T[...], x) + ib1[...], 0.0)
    inter = _bmm(iW2T[...], h1) + ib2[...]
    g1 = g1p + gWtT[...] * tT[...] + gb1[...]
    g1 = jnp.maximum(g1, 0.0)
    gate = _bmm(gW2T[...], g1) + gb2[...]
    w = inter * jax.nn.sigmoid(gate)
    d = dT[...]
    out[...] = w * relT[...] / d / (d + 1.0)


def _edge_dense(heT, nfiT, tT, relT, dT, Wcat, iW1T, ib1, iW2T, ib2,
                gWtT, gb1, gW2T, gb2):
    BE = 16000
    grid = (E // BE,)
    blk = lambda r: pl.BlockSpec((r, BE), lambda i: (0, i))
    full = lambda shape: pl.BlockSpec(shape, lambda i: (0, 0))
    return pl.pallas_call(
        _edge_body,
        grid=grid,
        in_specs=[
            blk(EDGE_DIM), blk(EDGE_DIM), blk(1), blk(3), blk(1),
            full((2 * NODE_DIM + 32, 2 * EDGE_DIM)),
            full((NODE_DIM, NODE_DIM)), full((NODE_DIM, 1)),
            full((1, NODE_DIM)), full((1, 1)),
            full((32, 1)), full((32, 1)), full((1, 32)), full((1, 1)),
        ],
        out_specs=pl.BlockSpec((3, BE), lambda i: (0, i)),
        out_shape=jax.ShapeDtypeStruct((3, E), jnp.float32),
    )(heT, nfiT, tT, relT, dT, Wcat, iW1T, ib1, iW2T, ib2,
      gWtT, gb1, gW2T, gb2)


NZC = 2000
NZS = N // NZC


@functools.partial(
    pl.kernel,
    out_type=jax.ShapeDtypeStruct((NC, 3, N), jnp.float32),
    mesh=plsc.VectorSubcoreMesh(**_SC_MESH),
    compiler_params=pltpu.CompilerParams(use_tc_tiling_on_sc=False),
    scratch_types=[
        pltpu.VMEM((GPC, GB), jnp.int32),
        pltpu.VMEM((3, CHUNK), jnp.float32),
        pltpu.VMEM((NZC,), jnp.float32),
        pltpu.VMEM_SHARED((N,), jnp.float32),
        pltpu.VMEM_SHARED((N,), jnp.float32),
        pltpu.VMEM_SHARED((N,), jnp.float32),
        pltpu.SemaphoreType.DMA,
    ],
)
def _sc_scatter(fT_hbm, li2_hbm, out, idxv, fcomp, zbuf, accx, accy, accz, sem):
    c = lax.axis_index("c")
    s = lax.axis_index("s")
    planes = [accx, accy, accz]

    def zrow(i, carry):
        zbuf[pl.ds(i * 16, 16)] = jnp.zeros((16,), jnp.float32)
        return carry

    lax.fori_loop(0, NZC // 16, zrow, 0)
    for comp in range(3):
        @pl.when(s == comp)
        def _(comp=comp):
            for j in range(NZS):
                pltpu.sync_copy(zbuf, planes[comp].at[pl.ds(j * NZC, NZC)])

    plsc.subcore_barrier()

    base0 = (c * NS + s) * EPW
    rbase0 = base0 // GB

    def step(i, carry):
        base = base0 + i * CHUNK
        rbase = rbase0 + i * GPC
        pltpu.sync_copy(li2_hbm.at[pl.ds(rbase, GPC)], idxv)
        for comp in range(3):
            pltpu.sync_copy(fT_hbm.at[comp, pl.ds(base, CHUNK)],
                            fcomp.at[comp])
        descs = []
        for comp in range(3):
            for j in range(GPC):
                descs.append(pltpu.async_copy(
                    fcomp.at[comp, pl.ds(j * GB, GB)],
                    planes[comp].at[idxv.at[j]], sem, add=True))
        for dsc in descs:
            dsc.wait()
        return carry

    lax.fori_loop(0, STEPS, step, 0)
    plsc.subcore_barrier()
    for comp in range(3):
        for j in range(NZS):
            @pl.when(s == comp * NZS + j)
            def _(comp=comp, j=j):
                pltpu.sync_copy(planes[comp].at[pl.ds(j * NZC, NZC)],
                                out.at[c, comp, pl.ds(j * NZC, NZC)])


def _comb_body(p, o):
    o[...] = p[0] + p[1]


def _combine(partials):
    return pl.pallas_call(
        _comb_body,
        out_shape=jax.ShapeDtypeStruct((3, N), jnp.float32),
    )(partials)


def kernel(h_node, h_edge, edge_index, relative_vec, distance, edge_time,
           left_W1, left_b1, left_W2, left_b2,
           right_W1, right_b1, right_W2, right_b2,
           bond_Wb, bond_Wn,
           inter_W1, inter_b1, inter_W2, inter_b2,
           gate_W1, gate_b1, gate_W2, gate_b2):
    li = edge_index[0]
    ri = edge_index[1]
    lproj, rproj = _node_proj(
        h_node,
        left_W1, left_b1.reshape(1, -1), left_W2, left_b2.reshape(1, -1),
        right_W1, right_b1.reshape(1, -1), right_W2, right_b2.reshape(1, -1))

    nfi = _sc_gather(lproj, rproj, li, ri)

    z16 = jnp.zeros((NODE_DIM, EDGE_DIM), jnp.float32)
    Wcat = jnp.concatenate([
        jnp.concatenate([bond_Wb.T, z16], axis=1),
        jnp.concatenate([z16, bond_Wn.T], axis=1),
        jnp.concatenate([gate_W1[:EDGE_DIM].T,
                         gate_W1[EDGE_DIM:2 * EDGE_DIM].T], axis=1),
    ], axis=0)

    forceT = _edge_dense(
        h_edge.T, nfi.astype(jnp.bfloat16).T, edge_time.reshape(1, E),
        relative_vec.T,
        distance.reshape(1, E),
        Wcat,
        inter_W1.T, inter_b1.reshape(-1, 1), inter_W2.reshape(1, -1),
        inter_b2.reshape(1, 1),
        gate_W1[2 * EDGE_DIM:].T, gate_b1.reshape(-1, 1),
        gate_W2.reshape(1, -1), gate_b2.reshape(1, 1))

    partials = _sc_scatter(forceT, li.reshape(E // GB, GB))
    return _combine(partials).T
